# baseline (speedup 1.0000x reference)
import jax
import jax.numpy as jnp
from jax import lax
from jax.experimental import pallas as pl
from jax.experimental.pallas import tpu as pltpu
from jax.experimental.pallas import tpu_sc as plsc
from functools import partial

PAD = 512
NW = 32
CH = 64


def _pack_bf16(v):
    h = v.shape[1] // 2
    u = lax.bitcast_convert_type(v, jnp.int32)
    r = (u + 0x7FFF + ((u >> 16) & 1)) >> 16
    lo = r[:, :h] & 0xFFFF
    hi = r[:, h:] << 16
    return lo | hi


def _unpack_bf16(w):
    lo = lax.bitcast_convert_type(w << 16, jnp.float32)
    hi = lax.bitcast_convert_type(w & jnp.int32(-65536), jnp.float32)
    return jnp.concatenate([lo, hi], axis=1).astype(jnp.bfloat16)


def _gate_body(x_ref, wg_ref, bg_ref, xpk_ref, s1_ref, s2_ref, w1_ref,
               w2_ref, wsel_ref, r1_ref, r2_ref, cnt_ref,
               tril_ref, *, n_experts):
    i = pl.program_id(0)
    xb = x_ref[...]
    logits = jnp.dot(xb, wg_ref[...], preferred_element_type=jnp.float32)
    logits = logits + bg_ref[...]
    bt = logits.shape[0]
    idx = lax.broadcasted_iota(jnp.int32, (bt, n_experts), 1)
    m1 = jnp.max(logits, axis=-1, keepdims=True)
    a1 = jnp.min(jnp.where(logits == m1, idx, n_experts), axis=-1, keepdims=True)
    sel1 = (idx == a1)
    masked = jnp.where(sel1, -jnp.inf, logits)
    m2 = jnp.max(masked, axis=-1, keepdims=True)
    a2 = jnp.min(jnp.where(masked == m2, idx, n_experts), axis=-1, keepdims=True)
    sel2 = (idx == a2)
    t = jnp.exp(m2 - m1)
    w1 = 1.0 / (1.0 + t)
    w2 = 1.0 - w1
    s1f = sel1.astype(jnp.float32)
    s2f = sel2.astype(jnp.float32)
    xpk_ref[...] = lax.bitcast_convert_type(_pack_bf16(xb), jnp.float32)
    s1_ref[...] = s1f
    s2_ref[...] = s2f
    w1_ref[...] = w1
    w2_ref[...] = w2
    wsel_ref[...] = s1f * w1 + s2f * w2

    @pl.when(i == 0)
    def _():
        cnt_ref[...] = jnp.zeros_like(cnt_ref)
        ri = lax.broadcasted_iota(jnp.int32, (bt, bt), 0)
        ci = lax.broadcasted_iota(jnp.int32, (bt, bt), 1)
        tril_ref[...] = (ri > ci).astype(jnp.bfloat16)

    sc = s1f + s2f
    rm = jnp.dot(tril_ref[...], sc.astype(jnp.bfloat16),
                 preferred_element_type=jnp.float32)
    rmc = rm + cnt_ref[...]
    r1_ref[...] = jnp.sum(s1f * rmc, axis=1, keepdims=True)
    r2_ref[...] = jnp.sum(s2f * rmc, axis=1, keepdims=True)
    cnt_ref[...] += jnp.sum(sc, axis=0, keepdims=True)


def _finalize_body(s1_ref, s2_ref, r1_ref, r2_ref, cnt_ref,
                   p1_ref, p2_ref, tmap_ref, offs_ref,
                   *, n_experts, n_tiles_max):
    i = pl.program_id(0)

    @pl.when(i == 0)
    def _():
        counts = cnt_ref[...]
        padded = jnp.ceil(counts / PAD) * PAD
        ti = lax.broadcasted_iota(jnp.int32, (n_experts, n_experts), 0)
        tj = lax.broadcasted_iota(jnp.int32, (n_experts, n_experts), 1)
        triu_inc = (ti <= tj).astype(jnp.float32)
        ends = jnp.dot(padded, triu_inc,
                       preferred_element_type=jnp.float32)
        offs_ref[...] = ends - padded
        r = (lax.broadcasted_iota(jnp.int32, (n_tiles_max, n_experts), 0)
             * PAD).astype(jnp.float32)
        texp = jnp.sum((r >= ends).astype(jnp.int32), axis=1, keepdims=True)
        total = jnp.sum(padded, axis=-1, keepdims=True)
        used = r[:, :1] < total
        tmap_ref[...] = jnp.where(used, jnp.minimum(texp, n_experts - 1), -1)

    offs = offs_ref[...]
    p1 = r1_ref[...] + jnp.sum(s1_ref[...] * offs, axis=1, keepdims=True)
    p2 = r2_ref[...] + jnp.sum(s2_ref[...] * offs, axis=1, keepdims=True)
    p1_ref[...] = p1.astype(jnp.int32)
    p2_ref[...] = p2.astype(jnp.int32)


def _gmm_body(tm_ref, xs_ref, we_ref, ys_ref):
    i = pl.program_id(0)

    @pl.when(tm_ref[i] >= 0)
    def _():
        xw = _unpack_bf16(lax.bitcast_convert_type(xs_ref[...], jnp.int32))
        acc = jnp.dot(xw, we_ref[0], preferred_element_type=jnp.float32)
        ys_ref[...] = lax.bitcast_convert_type(_pack_bf16(acc), jnp.float32)


def _combine_body(y1_ref, y2_ref, w1_ref, w2_ref, wsel_ref, be_ref, out_ref):
    y1 = _unpack_bf16(lax.bitcast_convert_type(y1_ref[...], jnp.int32))
    y2 = _unpack_bf16(lax.bitcast_convert_type(y2_ref[...], jnp.int32))
    bias = jnp.dot(wsel_ref[...], be_ref[...], preferred_element_type=jnp.float32)
    out_ref[...] = (w1_ref[...] * y1.astype(jnp.float32)
                    + w2_ref[...] * y2.astype(jnp.float32) + bias)


def _dispatch(xpk, pos_k3, np_rows):
    hp = xpk.shape[1]
    nc = xpk.shape[0] // (NW * CH)
    mesh = plsc.VectorSubcoreMesh(core_axis_name="c", subcore_axis_name="s")

    @partial(pl.kernel, mesh=mesh,
             out_type=jax.ShapeDtypeStruct((np_rows, hp), jnp.float32),
             scratch_types=[pltpu.VMEM((CH, hp), jnp.float32),
                            pltpu.VMEM((CH,), jnp.int32),
                            pltpu.VMEM((CH,), jnp.int32),
                            pltpu.SemaphoreType.DMA])
    def k(xpk_hbm, pos_hbm, xs_hbm, rows_v, idx0_v, idx1_v, sem):
        wid = lax.axis_index("s") * 2 + lax.axis_index("c")
        for c in range(nc):
            base = wid * (nc * CH) + c * CH
            pltpu.sync_copy(xpk_hbm.at[pl.ds(base, CH)], rows_v)
            pltpu.sync_copy(pos_hbm.at[wid * (nc * 2) + c * 2], idx0_v)
            pltpu.sync_copy(pos_hbm.at[wid * (nc * 2) + c * 2 + 1], idx1_v)
            cp0 = pltpu.async_copy(rows_v, xs_hbm.at[idx0_v], sem)
            cp1 = pltpu.async_copy(rows_v, xs_hbm.at[idx1_v], sem)
            cp0.wait()
            cp1.wait()

    return k(xpk, pos_k3)


def _unsort(ys, pos_k3, b_tokens):
    hp = ys.shape[1]
    nc = b_tokens // (NW * CH)
    mesh = plsc.VectorSubcoreMesh(core_axis_name="c", subcore_axis_name="s")
    oty = jax.ShapeDtypeStruct((b_tokens, hp), jnp.float32)

    @partial(pl.kernel, mesh=mesh, out_type=[oty, oty],
             scratch_types=[pltpu.VMEM((CH, hp), jnp.float32),
                            pltpu.VMEM((CH, hp), jnp.float32),
                            pltpu.VMEM((CH,), jnp.int32),
                            pltpu.VMEM((CH,), jnp.int32),
                            pltpu.SemaphoreType.DMA,
                            pltpu.SemaphoreType.DMA])
    def k(ys_hbm, pos_hbm, o1_hbm, o2_hbm, r0_v, r1_v, idx0_v, idx1_v,
          sem0, sem1):
        wid = lax.axis_index("s") * 2 + lax.axis_index("c")
        for c in range(nc):
            base = wid * (nc * CH) + c * CH
            pltpu.sync_copy(pos_hbm.at[wid * (nc * 2) + c * 2], idx0_v)
            pltpu.sync_copy(pos_hbm.at[wid * (nc * 2) + c * 2 + 1], idx1_v)
            cp0 = pltpu.async_copy(ys_hbm.at[idx0_v], r0_v, sem0)
            cp1 = pltpu.async_copy(ys_hbm.at[idx1_v], r1_v, sem1)
            cp0.wait()
            st0 = pltpu.async_copy(r0_v, o1_hbm.at[pl.ds(base, CH)], sem0)
            cp1.wait()
            st1 = pltpu.async_copy(r1_v, o2_hbm.at[pl.ds(base, CH)], sem1)
            st0.wait()
            st1.wait()

    return k(ys, pos_k3)


def kernel(x, Wg, bg, We, be):
    B, D = x.shape
    E, _, O = We.shape
    BG = 512
    HP = D // 2
    HO = O // 2
    nb = B // BG
    np_rows = 2 * B + E * PAD
    n_tiles = np_rows // PAD
    we_bf = We.astype(jnp.bfloat16)
    bg2 = bg.reshape(1, E)

    xpk, s1, s2, w1, w2, wsel, r1, r2, cnt = pl.pallas_call(
        partial(_gate_body, n_experts=E),
        grid=(nb,),
        in_specs=[
            pl.BlockSpec((BG, D), lambda i: (i, 0)),
            pl.BlockSpec((D, E), lambda i: (0, 0)),
            pl.BlockSpec((1, E), lambda i: (0, 0)),
        ],
        out_specs=[
            pl.BlockSpec((BG, HP), lambda i: (i, 0)),
            pl.BlockSpec((BG, E), lambda i: (i, 0)),
            pl.BlockSpec((BG, E), lambda i: (i, 0)),
            pl.BlockSpec((BG, 1), lambda i: (i, 0)),
            pl.BlockSpec((BG, 1), lambda i: (i, 0)),
            pl.BlockSpec((BG, E), lambda i: (i, 0)),
            pl.BlockSpec((BG, 1), lambda i: (i, 0)),
            pl.BlockSpec((BG, 1), lambda i: (i, 0)),
            pl.BlockSpec((1, E), lambda i: (0, 0)),
        ],
        out_shape=[
            jax.ShapeDtypeStruct((B, HP), jnp.float32),
            jax.ShapeDtypeStruct((B, E), jnp.float32),
            jax.ShapeDtypeStruct((B, E), jnp.float32),
            jax.ShapeDtypeStruct((B, 1), jnp.float32),
            jax.ShapeDtypeStruct((B, 1), jnp.float32),
            jax.ShapeDtypeStruct((B, E), jnp.float32),
            jax.ShapeDtypeStruct((B, 1), jnp.float32),
            jax.ShapeDtypeStruct((B, 1), jnp.float32),
            jax.ShapeDtypeStruct((1, E), jnp.float32),
        ],
        scratch_shapes=[pltpu.VMEM((BG, BG), jnp.bfloat16)],
        compiler_params=pltpu.CompilerParams(dimension_semantics=("arbitrary",)),
    )(x, Wg, bg2)

    pos1, pos2, tmap = pl.pallas_call(
        partial(_finalize_body, n_experts=E, n_tiles_max=64),
        grid=(nb,),
        in_specs=[
            pl.BlockSpec((BG, E), lambda i: (i, 0)),
            pl.BlockSpec((BG, E), lambda i: (i, 0)),
            pl.BlockSpec((BG, 1), lambda i: (i, 0)),
            pl.BlockSpec((BG, 1), lambda i: (i, 0)),
            pl.BlockSpec((1, E), lambda i: (0, 0)),
        ],
        out_specs=[
            pl.BlockSpec((BG, 1), lambda i: (i, 0)),
            pl.BlockSpec((BG, 1), lambda i: (i, 0)),
            pl.BlockSpec((64, 1), lambda i: (0, 0)),
        ],
        out_shape=[
            jax.ShapeDtypeStruct((B, 1), jnp.int32),
            jax.ShapeDtypeStruct((B, 1), jnp.int32),
            jax.ShapeDtypeStruct((64, 1), jnp.int32),
        ],
        scratch_shapes=[
            pltpu.VMEM((1, E), jnp.float32),
        ],
        compiler_params=pltpu.CompilerParams(dimension_semantics=("arbitrary",)),
    )(s1, s2, r1, r2, cnt)

    tmap40 = tmap.reshape(64)[:n_tiles]
    nc = B // (NW * CH)
    pos_k3 = (jnp.stack([pos1.reshape(NW, nc, CH), pos2.reshape(NW, nc, CH)],
                        axis=2).reshape(NW * nc * 2, CH))

    xs = _dispatch(xpk, pos_k3, np_rows)

    ys = pl.pallas_call(
        _gmm_body,
        grid_spec=pltpu.PrefetchScalarGridSpec(
            num_scalar_prefetch=1,
            grid=(n_tiles,),
            in_specs=[
                pl.BlockSpec((PAD, HP), lambda i, tm: (i, 0)),
                pl.BlockSpec((1, D, O), lambda i, tm: (jnp.maximum(tm[i], 0), 0, 0)),
            ],
            out_specs=pl.BlockSpec((PAD, HO), lambda i, tm: (i, 0)),
        ),
        out_shape=jax.ShapeDtypeStruct((np_rows, HO), jnp.float32),
        compiler_params=pltpu.CompilerParams(dimension_semantics=("arbitrary",)),
    )(tmap40, xs, we_bf)

    y1, y2 = _unsort(ys, pos_k3, B)

    return pl.pallas_call(
        _combine_body,
        grid=(nb,),
        in_specs=[
            pl.BlockSpec((BG, HO), lambda i: (i, 0)),
            pl.BlockSpec((BG, HO), lambda i: (i, 0)),
            pl.BlockSpec((BG, 1), lambda i: (i, 0)),
            pl.BlockSpec((BG, 1), lambda i: (i, 0)),
            pl.BlockSpec((BG, E), lambda i: (i, 0)),
            pl.BlockSpec((E, O), lambda i: (0, 0)),
        ],
        out_specs=pl.BlockSpec((BG, O), lambda i: (i, 0)),
        out_shape=jax.ShapeDtypeStruct((B, O), jnp.float32),
        compiler_params=pltpu.CompilerParams(dimension_semantics=("arbitrary",)),
    )(y1, y2, w1, w2, wsel, be)


# speedup vs baseline: 1.0234x; 1.0234x over previous
import jax
import jax.numpy as jnp
from jax import lax
from jax.experimental import pallas as pl
from jax.experimental.pallas import tpu as pltpu
from jax.experimental.pallas import tpu_sc as plsc
from functools import partial

PAD = 512
NW = 32
CH = 128


def _pack_bf16(v):
    h = v.shape[1] // 2
    u = lax.bitcast_convert_type(v, jnp.int32)
    r = (u + 0x7FFF + ((u >> 16) & 1)) >> 16
    lo = r[:, :h] & 0xFFFF
    hi = r[:, h:] << 16
    return lo | hi


def _unpack_bf16(w):
    lo = lax.bitcast_convert_type(w << 16, jnp.float32)
    hi = lax.bitcast_convert_type(w & jnp.int32(-65536), jnp.float32)
    return jnp.concatenate([lo, hi], axis=1).astype(jnp.bfloat16)


def _gate_body(x_ref, wg_ref, bg_ref, xpk_ref, s1_ref, s2_ref, w1_ref,
               w2_ref, wsel_ref, r1_ref, r2_ref, cnt_ref,
               tril_ref, *, n_experts):
    i = pl.program_id(0)
    xb = x_ref[...]
    logits = jnp.dot(xb, wg_ref[...], preferred_element_type=jnp.float32)
    logits = logits + bg_ref[...]
    bt = logits.shape[0]
    idx = lax.broadcasted_iota(jnp.int32, (bt, n_experts), 1)
    m1 = jnp.max(logits, axis=-1, keepdims=True)
    a1 = jnp.min(jnp.where(logits == m1, idx, n_experts), axis=-1, keepdims=True)
    sel1 = (idx == a1)
    masked = jnp.where(sel1, -jnp.inf, logits)
    m2 = jnp.max(masked, axis=-1, keepdims=True)
    a2 = jnp.min(jnp.where(masked == m2, idx, n_experts), axis=-1, keepdims=True)
    sel2 = (idx == a2)
    t = jnp.exp(m2 - m1)
    w1 = 1.0 / (1.0 + t)
    w2 = 1.0 - w1
    s1f = sel1.astype(jnp.float32)
    s2f = sel2.astype(jnp.float32)
    xpk_ref[...] = lax.bitcast_convert_type(_pack_bf16(xb), jnp.float32)
    s1_ref[...] = s1f
    s2_ref[...] = s2f
    w1_ref[...] = w1
    w2_ref[...] = w2
    wsel_ref[...] = s1f * w1 + s2f * w2

    @pl.when(i == 0)
    def _():
        cnt_ref[...] = jnp.zeros_like(cnt_ref)
        ri = lax.broadcasted_iota(jnp.int32, (bt, bt), 0)
        ci = lax.broadcasted_iota(jnp.int32, (bt, bt), 1)
        tril_ref[...] = (ri > ci).astype(jnp.bfloat16)

    sc = s1f + s2f
    rm = jnp.dot(tril_ref[...], sc.astype(jnp.bfloat16),
                 preferred_element_type=jnp.float32)
    rmc = rm + cnt_ref[...]
    r1_ref[...] = jnp.sum(s1f * rmc, axis=1, keepdims=True)
    r2_ref[...] = jnp.sum(s2f * rmc, axis=1, keepdims=True)
    cnt_ref[...] += jnp.sum(sc, axis=0, keepdims=True)


def _finalize_body(s1_ref, s2_ref, r1_ref, r2_ref, cnt_ref,
                   p1_ref, p2_ref, tmap_ref, offs_ref,
                   *, n_experts, n_tiles_max):
    i = pl.program_id(0)

    @pl.when(i == 0)
    def _():
        counts = cnt_ref[...]
        padded = jnp.ceil(counts / PAD) * PAD
        ti = lax.broadcasted_iota(jnp.int32, (n_experts, n_experts), 0)
        tj = lax.broadcasted_iota(jnp.int32, (n_experts, n_experts), 1)
        triu_inc = (ti <= tj).astype(jnp.float32)
        ends = jnp.dot(padded, triu_inc,
                       preferred_element_type=jnp.float32)
        offs_ref[...] = ends - padded
        r = (lax.broadcasted_iota(jnp.int32, (n_tiles_max, n_experts), 0)
             * PAD).astype(jnp.float32)
        texp = jnp.sum((r >= ends).astype(jnp.int32), axis=1, keepdims=True)
        total = jnp.sum(padded, axis=-1, keepdims=True)
        used = r[:, :1] < total
        tmap_ref[...] = jnp.where(used, jnp.minimum(texp, n_experts - 1), -1)

    offs = offs_ref[...]
    p1 = r1_ref[...] + jnp.sum(s1_ref[...] * offs, axis=1, keepdims=True)
    p2 = r2_ref[...] + jnp.sum(s2_ref[...] * offs, axis=1, keepdims=True)
    p1_ref[...] = p1.astype(jnp.int32)
    p2_ref[...] = p2.astype(jnp.int32)


def _gmm_body(tm_ref, xs_ref, we_ref, ys_ref):
    i = pl.program_id(0)

    @pl.when(tm_ref[i] >= 0)
    def _():
        xw = _unpack_bf16(lax.bitcast_convert_type(xs_ref[...], jnp.int32))
        acc = jnp.dot(xw, we_ref[0], preferred_element_type=jnp.float32)
        ys_ref[...] = lax.bitcast_convert_type(_pack_bf16(acc), jnp.float32)


def _combine_body(y1_ref, y2_ref, w1_ref, w2_ref, wsel_ref, be_ref, out_ref):
    y1 = _unpack_bf16(lax.bitcast_convert_type(y1_ref[...], jnp.int32))
    y2 = _unpack_bf16(lax.bitcast_convert_type(y2_ref[...], jnp.int32))
    bias = jnp.dot(wsel_ref[...], be_ref[...], preferred_element_type=jnp.float32)
    out_ref[...] = (w1_ref[...] * y1.astype(jnp.float32)
                    + w2_ref[...] * y2.astype(jnp.float32) + bias)


def _dispatch(xpk, pos_k3, np_rows):
    hp = xpk.shape[1]
    nc = xpk.shape[0] // (NW * CH)
    mesh = plsc.VectorSubcoreMesh(core_axis_name="c", subcore_axis_name="s")

    @partial(pl.kernel, mesh=mesh,
             out_type=jax.ShapeDtypeStruct((np_rows, hp), jnp.float32),
             scratch_types=[pltpu.VMEM((CH, hp), jnp.float32),
                            pltpu.VMEM((CH,), jnp.int32),
                            pltpu.VMEM((CH,), jnp.int32),
                            pltpu.SemaphoreType.DMA])
    def k(xpk_hbm, pos_hbm, xs_hbm, rows_v, idx0_v, idx1_v, sem):
        wid = lax.axis_index("s") * 2 + lax.axis_index("c")
        for c in range(nc):
            base = wid * (nc * CH) + c * CH
            pltpu.sync_copy(xpk_hbm.at[pl.ds(base, CH)], rows_v)
            pltpu.sync_copy(pos_hbm.at[wid * (nc * 2) + c * 2], idx0_v)
            pltpu.sync_copy(pos_hbm.at[wid * (nc * 2) + c * 2 + 1], idx1_v)
            cp0 = pltpu.async_copy(rows_v, xs_hbm.at[idx0_v], sem)
            cp1 = pltpu.async_copy(rows_v, xs_hbm.at[idx1_v], sem)
            cp0.wait()
            cp1.wait()

    return k(xpk, pos_k3)


def _unsort(ys, pos_k3, b_tokens):
    hp = ys.shape[1]
    nc = b_tokens // (NW * CH)
    mesh = plsc.VectorSubcoreMesh(core_axis_name="c", subcore_axis_name="s")
    oty = jax.ShapeDtypeStruct((b_tokens, hp), jnp.float32)

    @partial(pl.kernel, mesh=mesh, out_type=[oty, oty],
             scratch_types=[pltpu.VMEM((CH, hp), jnp.float32),
                            pltpu.VMEM((CH,), jnp.int32),
                            pltpu.SemaphoreType.DMA])
    def k(ys_hbm, pos_hbm, o1_hbm, o2_hbm, rows_v, idx_v, sem):
        wid = lax.axis_index("s") * 2 + lax.axis_index("c")
        outs = (o1_hbm, o2_hbm)
        for c in range(nc):
            base = wid * (nc * CH) + c * CH
            for s in range(2):
                pltpu.sync_copy(pos_hbm.at[wid * (nc * 2) + c * 2 + s], idx_v)
                pltpu.async_copy(ys_hbm.at[idx_v], rows_v, sem).wait()
                pltpu.sync_copy(rows_v, outs[s].at[pl.ds(base, CH)])

    return k(ys, pos_k3)


def kernel(x, Wg, bg, We, be):
    B, D = x.shape
    E, _, O = We.shape
    BG = 512
    HP = D // 2
    HO = O // 2
    nb = B // BG
    np_rows = 2 * B + E * PAD
    n_tiles = np_rows // PAD
    we_bf = We.astype(jnp.bfloat16)
    bg2 = bg.reshape(1, E)

    xpk, s1, s2, w1, w2, wsel, r1, r2, cnt = pl.pallas_call(
        partial(_gate_body, n_experts=E),
        grid=(nb,),
        in_specs=[
            pl.BlockSpec((BG, D), lambda i: (i, 0)),
            pl.BlockSpec((D, E), lambda i: (0, 0)),
            pl.BlockSpec((1, E), lambda i: (0, 0)),
        ],
        out_specs=[
            pl.BlockSpec((BG, HP), lambda i: (i, 0)),
            pl.BlockSpec((BG, E), lambda i: (i, 0)),
            pl.BlockSpec((BG, E), lambda i: (i, 0)),
            pl.BlockSpec((BG, 1), lambda i: (i, 0)),
            pl.BlockSpec((BG, 1), lambda i: (i, 0)),
            pl.BlockSpec((BG, E), lambda i: (i, 0)),
            pl.BlockSpec((BG, 1), lambda i: (i, 0)),
            pl.BlockSpec((BG, 1), lambda i: (i, 0)),
            pl.BlockSpec((1, E), lambda i: (0, 0)),
        ],
        out_shape=[
            jax.ShapeDtypeStruct((B, HP), jnp.float32),
            jax.ShapeDtypeStruct((B, E), jnp.float32),
            jax.ShapeDtypeStruct((B, E), jnp.float32),
            jax.ShapeDtypeStruct((B, 1), jnp.float32),
            jax.ShapeDtypeStruct((B, 1), jnp.float32),
            jax.ShapeDtypeStruct((B, E), jnp.float32),
            jax.ShapeDtypeStruct((B, 1), jnp.float32),
            jax.ShapeDtypeStruct((B, 1), jnp.float32),
            jax.ShapeDtypeStruct((1, E), jnp.float32),
        ],
        scratch_shapes=[pltpu.VMEM((BG, BG), jnp.bfloat16)],
        compiler_params=pltpu.CompilerParams(dimension_semantics=("arbitrary",)),
    )(x, Wg, bg2)

    pos1, pos2, tmap = pl.pallas_call(
        partial(_finalize_body, n_experts=E, n_tiles_max=64),
        grid=(nb,),
        in_specs=[
            pl.BlockSpec((BG, E), lambda i: (i, 0)),
            pl.BlockSpec((BG, E), lambda i: (i, 0)),
            pl.BlockSpec((BG, 1), lambda i: (i, 0)),
            pl.BlockSpec((BG, 1), lambda i: (i, 0)),
            pl.BlockSpec((1, E), lambda i: (0, 0)),
        ],
        out_specs=[
            pl.BlockSpec((BG, 1), lambda i: (i, 0)),
            pl.BlockSpec((BG, 1), lambda i: (i, 0)),
            pl.BlockSpec((64, 1), lambda i: (0, 0)),
        ],
        out_shape=[
            jax.ShapeDtypeStruct((B, 1), jnp.int32),
            jax.ShapeDtypeStruct((B, 1), jnp.int32),
            jax.ShapeDtypeStruct((64, 1), jnp.int32),
        ],
        scratch_shapes=[
            pltpu.VMEM((1, E), jnp.float32),
        ],
        compiler_params=pltpu.CompilerParams(dimension_semantics=("arbitrary",)),
    )(s1, s2, r1, r2, cnt)

    tmap40 = tmap.reshape(64)[:n_tiles]
    nc = B // (NW * CH)
    pos_k3 = (jnp.stack([pos1.reshape(NW, nc, CH), pos2.reshape(NW, nc, CH)],
                        axis=2).reshape(NW * nc * 2, CH))

    xs = _dispatch(xpk, pos_k3, np_rows)

    ys = pl.pallas_call(
        _gmm_body,
        grid_spec=pltpu.PrefetchScalarGridSpec(
            num_scalar_prefetch=1,
            grid=(n_tiles,),
            in_specs=[
                pl.BlockSpec((PAD, HP), lambda i, tm: (i, 0)),
                pl.BlockSpec((1, D, O), lambda i, tm: (jnp.maximum(tm[i], 0), 0, 0)),
            ],
            out_specs=pl.BlockSpec((PAD, HO), lambda i, tm: (i, 0)),
        ),
        out_shape=jax.ShapeDtypeStruct((np_rows, HO), jnp.float32),
        compiler_params=pltpu.CompilerParams(dimension_semantics=("arbitrary",)),
    )(tmap40, xs, we_bf)

    y1, y2 = _unsort(ys, pos_k3, B)

    return pl.pallas_call(
        _combine_body,
        grid=(nb,),
        in_specs=[
            pl.BlockSpec((BG, HO), lambda i: (i, 0)),
            pl.BlockSpec((BG, HO), lambda i: (i, 0)),
            pl.BlockSpec((BG, 1), lambda i: (i, 0)),
            pl.BlockSpec((BG, 1), lambda i: (i, 0)),
            pl.BlockSpec((BG, E), lambda i: (i, 0)),
            pl.BlockSpec((E, O), lambda i: (0, 0)),
        ],
        out_specs=pl.BlockSpec((BG, O), lambda i: (i, 0)),
        out_shape=jax.ShapeDtypeStruct((B, O), jnp.float32),
        compiler_params=pltpu.CompilerParams(dimension_semantics=("arbitrary",)),
    )(y1, y2, w1, w2, wsel, be)
